# baseline (speedup 1.0000x reference)
import functools

import jax
import jax.numpy as jnp
from jax import lax
from jax.experimental import pallas as pl
from jax.experimental.pallas import tpu as pltpu
from jax.experimental.pallas import tpu_sc as plsc

N = 10000
T = 3
E = 320000
D = 128

NC = 2
NS = 16
NW = NC * NS
CHUNK = 128

PAD_ROWS = 112
NP = N + PAD_ROWS
RPT = NP // NS
PAD_SPREAD = 48

EPW = E // NW
CPT = 80
MT = CPT * CHUNK
NPAD = MT - EPW
PAIRS = CPT // 2
PCHUNK = 64
NBUF = 4
PCPT = MT // PCHUNK
QUADS = PCPT // NBUF

TNP = T * NP

BN = 1000
NBLK = N // BN


@functools.cache
def _make_deg_kernel():
  mesh = plsc.VectorSubcoreMesh(core_axis_name="c", subcore_axis_name="s")

  @functools.partial(
      pl.kernel,
      mesh=mesh,
      out_type=jax.ShapeDtypeStruct((2 * TNP, D), jnp.float32),
      scratch_types=[
          pltpu.VMEM((CHUNK,), jnp.int32),
          pltpu.VMEM((CHUNK,), jnp.int32),
          pltpu.VMEM((CHUNK, D), jnp.float32),
          pltpu.VMEM_SHARED((NP, D), jnp.float32),
          pltpu.SemaphoreType.DMA,
          pltpu.SemaphoreType.DMA,
      ],
  )
  def _deg_kernel(dst_hbm, ones_hbm, zero_hbm, out, didx_a, didx_b, ones_v,
                  acc_sh, sem_a, sem_b):
    c = lax.axis_index("c")
    s = lax.axis_index("s")
    w = s * NC + c
    r0 = s * RPT
    pltpu.sync_copy(ones_hbm, ones_v)
    for t in range(T):
        ebase = (t * NW + w) * MT
        pltpu.sync_copy(zero_hbm, acc_sh.at[pl.ds(r0, RPT)])
        plsc.subcore_barrier()
        pltpu.async_copy(dst_hbm.at[pl.ds(ebase, CHUNK)], didx_a, sem_a)
        pltpu.async_copy(dst_hbm.at[pl.ds(ebase + CHUNK, CHUNK)], didx_b,
                         sem_b)

        def body(j, carry, _ebase=ebase):
            o = _ebase + j * (2 * CHUNK)
            pltpu.make_async_copy(dst_hbm.at[pl.ds(o, CHUNK)], didx_a,
                                  sem_a).wait()
            pltpu.sync_copy(ones_v, acc_sh.at[didx_a], add=True)

            @pl.when(j < PAIRS - 1)
            def _():
                pltpu.async_copy(dst_hbm.at[pl.ds(o + 2 * CHUNK, CHUNK)],
                                 didx_a, sem_a)

            pltpu.make_async_copy(dst_hbm.at[pl.ds(o + CHUNK, CHUNK)],
                                  didx_b, sem_b).wait()
            pltpu.sync_copy(ones_v, acc_sh.at[didx_b], add=True)

            @pl.when(j < PAIRS - 1)
            def _():
                pltpu.async_copy(dst_hbm.at[pl.ds(o + 3 * CHUNK, CHUNK)],
                                 didx_b, sem_b)

            return carry

        lax.fori_loop(0, PAIRS, body, 0)
        plsc.subcore_barrier()
        pltpu.sync_copy(acc_sh.at[pl.ds(r0, RPT)],
                        out.at[pl.ds(c * TNP + t * NP + r0, RPT)])

  return _deg_kernel


@functools.cache
def _make_prop_kernel():
  mesh = plsc.VectorSubcoreMesh(core_axis_name="c", subcore_axis_name="s")

  @functools.partial(
      pl.kernel,
      mesh=mesh,
      out_type=jax.ShapeDtypeStruct((2 * TNP, D), jnp.float32),
      scratch_types=(
          [pltpu.VMEM((MT,), jnp.int32)]
          + [pltpu.VMEM((PCHUNK,), jnp.int32) for _ in range(NBUF)]
          + [pltpu.VMEM((PCHUNK, D), jnp.float32) for _ in range(NBUF)]
          + [pltpu.SemaphoreType.DMA for _ in range(2 * NBUF)]
          + [pltpu.VMEM_SHARED((NP, D), jnp.float32)]
      ),
  )
  def _prop_kernel(tab_hbm, src_hbm, dst_hbm, zero_hbm, out, sall,
                   di0, di1, di2, di3, ro0, ro1, ro2, ro3,
                   sg0, sg1, sg2, sg3, sd0, sd1, sd2, sd3, acc_sh):
    didx = [di0, di1, di2, di3]
    rows = [ro0, ro1, ro2, ro3]
    semg = [sg0, sg1, sg2, sg3]
    semd = [sd0, sd1, sd2, sd3]
    c = lax.axis_index("c")
    s = lax.axis_index("s")
    w = s * NC + c
    r0 = s * RPT
    for t in range(T):
        ebase = (t * NW + w) * MT
        pltpu.async_copy(src_hbm.at[pl.ds(ebase, MT)], sall, sg0)
        pltpu.sync_copy(zero_hbm, acc_sh.at[pl.ds(r0, RPT)])
        pltpu.make_async_copy(src_hbm.at[pl.ds(ebase, MT)], sall, sg0).wait()
        plsc.subcore_barrier()

        for k in range(NBUF):
            pltpu.async_copy(tab_hbm.at[sall.at[pl.ds(k * PCHUNK, PCHUNK)]],
                             rows[k], semg[k])
            pltpu.async_copy(dst_hbm.at[pl.ds(ebase + k * PCHUNK, PCHUNK)],
                             didx[k], semd[k])

        def body(j, carry, _ebase=ebase):
            o = j * (NBUF * PCHUNK)
            for k in range(NBUF):
                ck = o + k * PCHUNK
                pltpu.make_async_copy(
                    dst_hbm.at[pl.ds(_ebase + ck, PCHUNK)], didx[k],
                    semd[k]).wait()
                pltpu.make_async_copy(
                    tab_hbm.at[sall.at[pl.ds(ck, PCHUNK)]], rows[k],
                    semg[k]).wait()
                pltpu.sync_copy(rows[k], acc_sh.at[didx[k]], add=True)

                @pl.when(j < QUADS - 1)
                def _(k=k, ck=ck):
                    nk = ck + NBUF * PCHUNK
                    pltpu.async_copy(
                        tab_hbm.at[sall.at[pl.ds(nk, PCHUNK)]], rows[k],
                        semg[k])
                    pltpu.async_copy(
                        dst_hbm.at[pl.ds(_ebase + nk, PCHUNK)], didx[k],
                        semd[k])

            return carry

        lax.fori_loop(0, QUADS, body, 0)
        plsc.subcore_barrier()
        pltpu.sync_copy(acc_sh.at[pl.ds(r0, RPT)],
                        out.at[pl.ds(c * TNP + t * NP + r0, RPT)])

  return _prop_kernel


def _dinv_from(d0, d1):
    deg = 1.0 + d0[..., 0:1] + d1[..., 0:1]
    return lax.rsqrt(deg)


def _prescale_body(x_ref, w_ref, d0_ref, d1_ref, o_ref, dv_ref):
    dinv = _dinv_from(d0_ref[0, 0], d1_ref[0, 0])
    h = jnp.dot(x_ref[0], w_ref[0], preferred_element_type=jnp.float32)
    o_ref[0] = h * dinv
    dv_ref[0] = jnp.broadcast_to(dinv, (BN, 16))


def _layer2_body(s0_ref, s1_ref, hp_ref, dv_ref, w_ref, b_ref, o_ref):
    dinv = dv_ref[0][:, 0:1]
    pre = (s0_ref[0, 0] + s1_ref[0, 0] + hp_ref[0]) * dinv + b_ref[0, 0]
    h1 = jnp.maximum(pre, 0.0)
    g = jnp.dot(h1, w_ref[0], preferred_element_type=jnp.float32)
    o_ref[0] = g * dinv


def _gru_body(s0_ref, s1_ref, gp_ref, dv_ref, b2_ref, wih_ref,
              whh_ref, bih_ref, bhh_ref, gam_ref, bet_ref, o_ref):
    hx = []
    for t in range(T):
        dinv = dv_ref[t][:, 0:1]
        x2 = (s0_ref[0, t] + s1_ref[0, t] + gp_ref[t]) * dinv + b2_ref[t]
        nrm = jnp.sqrt(jnp.sum(x2 * x2, axis=1, keepdims=True))
        hx.append(x2 / jnp.maximum(nrm, 1e-12))
    h = jnp.zeros_like(hx[0])
    gam = gam_ref[0]
    bet = bet_ref[0]
    wih = wih_ref[...]
    whh = whh_ref[...]
    for t in range(T):
        gx = jnp.dot(hx[t], wih, preferred_element_type=jnp.float32) + bih_ref[0]
        gh = jnp.dot(h, whh, preferred_element_type=jnp.float32) + bhh_ref[0]
        r = jax.nn.sigmoid(gx[:, :D] + gh[:, :D])
        z = jax.nn.sigmoid(gx[:, D:2 * D] + gh[:, D:2 * D])
        n = jnp.tanh(gx[:, 2 * D:] + r * gh[:, 2 * D:])
        h = (1.0 - z) * n + z * h
        mu = jnp.mean(h, axis=1, keepdims=True)
        var = jnp.mean((h - mu) * (h - mu), axis=1, keepdims=True)
        o_ref[t] = gam * (h - mu) * lax.rsqrt(var + 1e-5) + bet


def kernel(x_list, edge_list, gcn_w1, gcn_b1, gcn_w2, gcn_b2, gru_w_ih,
           gru_w_hh, gru_b_ih, gru_b_hh, ln_gamma, ln_beta):
    f32 = jnp.float32

    src = edge_list[:, 0, :].reshape(T, NW, EPW)
    dst = edge_list[:, 1, :].reshape(T, NW, EPW)
    padv = (N + (jnp.arange(NPAD, dtype=jnp.int32) % PAD_SPREAD))
    pad3 = jnp.broadcast_to(padv, (T, NW, NPAD))
    src_p = jnp.concatenate([src, pad3], axis=2)
    dst_p = jnp.concatenate([dst, pad3], axis=2)
    tshift = (jnp.arange(T, dtype=jnp.int32) * NP)[:, None, None]
    src_g = (src_p + tshift).reshape(-1)
    dst_f = dst_p.reshape(-1)

    ones_rows = jnp.ones((CHUNK, D), f32)
    zero_rows = jnp.zeros((RPT, D), f32)

    degs = _make_deg_kernel()(dst_f, ones_rows, zero_rows)
    degs = degs.reshape(2, T, NP, D)

    dspec = pl.BlockSpec((1, 1, BN, D), lambda t, i: (0, t, i, 0))
    dspec1 = pl.BlockSpec((1, 1, BN, D), lambda t, i: (1, t, i, 0))

    hp, dv = pl.pallas_call(
        _prescale_body,
        grid=(T, NBLK),
        in_specs=[
            pl.BlockSpec((1, BN, D), lambda t, i: (t, i, 0)),
            pl.BlockSpec((1, D, D), lambda t, i: (t, 0, 0)),
            dspec,
            dspec1,
        ],
        out_specs=[
            pl.BlockSpec((1, BN, D), lambda t, i: (t, i, 0)),
            pl.BlockSpec((1, BN, 16), lambda t, i: (t, i, 0)),
        ],
        out_shape=[
            jax.ShapeDtypeStruct((T, NP, D), f32),
            jax.ShapeDtypeStruct((T, NP, 16), f32),
        ],
    )(x_list, gcn_w1, degs, degs)

    s1 = _make_prop_kernel()(hp.reshape(TNP, D), src_g, dst_f, zero_rows)
    s1 = s1.reshape(2, T, NP, D)

    pspec0 = pl.BlockSpec((1, 1, BN, D), lambda t, i: (0, t, i, 0))
    pspec1 = pl.BlockSpec((1, 1, BN, D), lambda t, i: (1, t, i, 0))

    gp = pl.pallas_call(
        _layer2_body,
        grid=(T, NBLK),
        in_specs=[
            pspec0,
            pspec1,
            pl.BlockSpec((1, BN, D), lambda t, i: (t, i, 0)),
            pl.BlockSpec((1, BN, 16), lambda t, i: (t, i, 0)),
            pl.BlockSpec((1, D, D), lambda t, i: (t, 0, 0)),
            pl.BlockSpec((1, 1, D), lambda t, i: (t, 0, 0)),
        ],
        out_specs=pl.BlockSpec((1, BN, D), lambda t, i: (t, i, 0)),
        out_shape=jax.ShapeDtypeStruct((T, NP, D), f32),
    )(s1, s1, hp, dv, gcn_w2, gcn_b1.reshape(T, 1, D))

    s2 = _make_prop_kernel()(gp.reshape(TNP, D), src_g, dst_f, zero_rows)
    s2 = s2.reshape(2, T, NP, D)

    out = pl.pallas_call(
        _gru_body,
        grid=(NBLK,),
        in_specs=[
            pl.BlockSpec((1, T, BN, D), lambda i: (0, 0, i, 0)),
            pl.BlockSpec((1, T, BN, D), lambda i: (1, 0, i, 0)),
            pl.BlockSpec((T, BN, D), lambda i: (0, i, 0)),
            pl.BlockSpec((T, BN, 16), lambda i: (0, i, 0)),
            pl.BlockSpec((T, D), lambda i: (0, 0)),
            pl.BlockSpec((D, 3 * D), lambda i: (0, 0)),
            pl.BlockSpec((D, 3 * D), lambda i: (0, 0)),
            pl.BlockSpec((1, 3 * D), lambda i: (0, 0)),
            pl.BlockSpec((1, 3 * D), lambda i: (0, 0)),
            pl.BlockSpec((1, D), lambda i: (0, 0)),
            pl.BlockSpec((1, D), lambda i: (0, 0)),
        ],
        out_specs=pl.BlockSpec((T, BN, D), lambda i: (0, i, 0)),
        out_shape=jax.ShapeDtypeStruct((T, N, D), f32),
    )(s2, s2, gp, dv, gcn_b2, gru_w_ih.T, gru_w_hh.T,
      gru_b_ih.reshape(1, 3 * D), gru_b_hh.reshape(1, 3 * D),
      ln_gamma.reshape(1, D), ln_beta.reshape(1, D))

    return out
